# baseline (speedup 1.0000x reference)
import functools

import jax
import jax.numpy as jnp
from jax import lax
from jax.experimental import pallas as pl
from jax.experimental.pallas import tpu as pltpu
from jax.experimental.pallas import tpu_sc as plsc

NC = 2
NS = 16
NW = NC * NS
BATCH = 16384
BPW = BATCH // NW
ICH = 128
NIDX = BPW // ICH
W3MAX = 300000 // 8 - 1


@jax.jit
def _sc_gather_concat(intr8, rot8, trans8, idx2):
    mesh = plsc.VectorSubcoreMesh(core_axis_name="c", subcore_axis_name="s")

    @functools.partial(
        pl.kernel,
        mesh=mesh,
        out_type=jax.ShapeDtypeStruct((NW, BPW, 10), jnp.float32),
        compiler_params=pltpu.CompilerParams(
            use_tc_tiling_on_sc=False, needs_layout_passes=False),
        scratch_types=[
            pltpu.VMEM((BPW,), jnp.int32),
            pltpu.VMEM((NIDX, ICH), jnp.int32),
            pltpu.VMEM((NIDX, ICH), jnp.int32),
            pltpu.VMEM((NIDX, ICH), jnp.int32),
            pltpu.VMEM((BPW, 8), jnp.float32),
            pltpu.VMEM((2 * BPW, 8), jnp.float32),
            pltpu.VMEM((2 * BPW, 8), jnp.float32),
            pltpu.VMEM((BPW, 10), jnp.float32),
            pltpu.SemaphoreType.DMA,
        ],
    )
    def k(intr_hbm, rot_hbm, trans_hbm, idx_hbm, out_hbm,
          idx_v, il4, ilg, ilg1, v4, vr, vt, out_v, sem):
        wid = lax.axis_index("s") * NC + lax.axis_index("c")
        pltpu.sync_copy(idx_hbm.at[wid], idx_v)

        for ch in range(BPW // 16):
            j, off = divmod(ch * 16, ICH)
            iv = idx_v[pl.ds(ch * 16, 16)]
            il4[j, pl.ds(off, 16)] = lax.shift_right_logical(iv, 1)
            g = lax.shift_right_logical(iv * 3, 3)
            ilg[j, pl.ds(off, 16)] = g
            ilg1[j, pl.ds(off, 16)] = jnp.minimum(g + 1, W3MAX)

        copies = []
        for j in range(NIDX):
            r0 = j * ICH
            copies.append(pltpu.async_copy(
                intr_hbm.at[il4.at[j]], v4.at[pl.ds(r0, ICH), :], sem))
            copies.append(pltpu.async_copy(
                rot_hbm.at[ilg.at[j]], vr.at[pl.ds(r0, ICH), :], sem))
            copies.append(pltpu.async_copy(
                rot_hbm.at[ilg1.at[j]], vr.at[pl.ds(BPW + r0, ICH), :], sem))
            copies.append(pltpu.async_copy(
                trans_hbm.at[ilg.at[j]], vt.at[pl.ds(r0, ICH), :], sem))
            copies.append(pltpu.async_copy(
                trans_hbm.at[ilg1.at[j]], vt.at[pl.ds(BPW + r0, ICH), :], sem))
        for cpy in copies:
            cpy.wait()

        iota = lax.iota(jnp.int32, 16)

        def body(chunk, carry):
            rows = chunk * 16 + iota
            iv = idx_v[pl.ds(chunk * 16, 16)]
            colbase = (iv & 1) * 4
            p3 = (iv * 3) & 7
            for c in range(4):
                vals = plsc.load_gather(v4, [rows, colbase + c])
                plsc.store_scatter(out_v, [rows, jnp.full((16,), c, jnp.int32)],
                                   vals)
            for dc in range(3):
                pos = p3 + dc
                srow = rows + lax.shift_left(
                    lax.shift_right_logical(pos, 3), 9)
                scol = pos & 7
                vals = plsc.load_gather(vr, [srow, scol])
                plsc.store_scatter(
                    out_v, [rows, jnp.full((16,), 4 + dc, jnp.int32)], vals)
                vals = plsc.load_gather(vt, [srow, scol])
                plsc.store_scatter(
                    out_v, [rows, jnp.full((16,), 7 + dc, jnp.int32)], vals)
            return carry

        lax.fori_loop(0, BPW // 16, body, 0)
        pltpu.sync_copy(out_v, out_hbm.at[wid])

    return k(intr8, rot8, trans8, idx2)


def kernel(intrinsic_deltas, rotation_deltas, translation_deltas, camera_idx):
    intr8 = intrinsic_deltas.reshape(50000, 8)
    rot8 = rotation_deltas.reshape(37500, 8)
    trans8 = translation_deltas.reshape(37500, 8)
    idx2 = camera_idx.astype(jnp.int32).reshape(NW, BPW)
    out = _sc_gather_concat(intr8, rot8, trans8, idx2)
    return out.reshape(BATCH, 10)


# speedup vs baseline: 4.9588x; 4.9588x over previous
import functools

import jax
import jax.numpy as jnp
from jax import lax
from jax.experimental import pallas as pl
from jax.experimental.pallas import tpu as pltpu
from jax.experimental.pallas import tpu_sc as plsc

NC = 2
NS = 16
NW = NC * NS
BATCH = 16384
BPW = BATCH // NW
ICH = 128
NIDX = BPW // ICH
NROW = 100000
CW = NROW // 8


@jax.jit
def _sc_gather_concat(intr8, rot8, trans8, idx):
    mesh = plsc.VectorSubcoreMesh(core_axis_name="c", subcore_axis_name="s")

    @functools.partial(
        pl.kernel,
        mesh=mesh,
        out_type=jax.ShapeDtypeStruct((BATCH, 10), jnp.float32),
        compiler_params=pltpu.CompilerParams(
            use_tc_tiling_on_sc=False, needs_layout_passes=False),
        scratch_types=[
            pltpu.VMEM((BPW,), jnp.int32),
            pltpu.VMEM((NIDX, ICH), jnp.int32),
            pltpu.VMEM((10 * BPW, 8), jnp.float32),
            pltpu.VMEM((BPW, 10), jnp.float32),
            pltpu.SemaphoreType.DMA,
        ],
    )
    def k(intr_hbm, rot_hbm, trans_hbm, idx_hbm, out_hbm,
          idx_v, gl, vbuf, out_v, sem):
        wid = lax.axis_index("s") * NC + lax.axis_index("c")
        base = wid * BPW
        pltpu.sync_copy(idx_hbm.at[pl.ds(base, BPW)], idx_v)

        for ch in range(BPW // 16):
            j, off = divmod(ch * 16, ICH)
            iv = idx_v[pl.ds(ch * 16, 16)]
            gl[j, pl.ds(off, 16)] = lax.shift_right_logical(iv, 3)

        sources = ([intr_hbm.at[pl.ds(c * CW, CW), :] for c in range(4)]
                   + [rot_hbm.at[pl.ds(c * CW, CW), :] for c in range(3)]
                   + [trans_hbm.at[pl.ds(c * CW, CW), :] for c in range(3)])
        copies = []
        for q, src in enumerate(sources):
            for j in range(NIDX):
                copies.append(pltpu.async_copy(
                    src.at[gl.at[j]],
                    vbuf.at[pl.ds(q * BPW + j * ICH, ICH), :], sem))
        for cpy in copies:
            cpy.wait()

        iota = lax.iota(jnp.int32, 16)

        def body(chunk, carry):
            rows = chunk * 16 + iota
            iv = idx_v[pl.ds(chunk * 16, 16)]
            off = iv & 7
            for q in range(10):
                vals = plsc.load_gather(vbuf, [q * BPW + rows, off])
                plsc.store_scatter(out_v, [rows, jnp.full((16,), q, jnp.int32)],
                                   vals)
            return carry

        lax.fori_loop(0, BPW // 16, body, 0)
        pltpu.sync_copy(out_v, out_hbm.at[pl.ds(base, BPW), :])

    return k(intr8, rot8, trans8, idx)


def kernel(intrinsic_deltas, rotation_deltas, translation_deltas, camera_idx):
    intr8 = intrinsic_deltas.T.reshape(4 * NROW // 8, 8)
    rot8 = rotation_deltas.T.reshape(3 * NROW // 8, 8)
    trans8 = translation_deltas.T.reshape(3 * NROW // 8, 8)
    return _sc_gather_concat(intr8, rot8, trans8,
                             camera_idx.astype(jnp.int32))


# speedup vs baseline: 6.3709x; 1.2848x over previous
import functools

import jax
import jax.numpy as jnp
from jax import lax
from jax.experimental import pallas as pl
from jax.experimental.pallas import tpu as pltpu
from jax.experimental.pallas import tpu_sc as plsc

NC = 2
NS = 16
NW = NC * NS
BATCH = 16384
BPW = BATCH // NW
NROW = 100000
CW = NROW // 8


@jax.jit
def _sc_gather_concat(intr8, rot8, trans8, idx):
    mesh = plsc.VectorSubcoreMesh(core_axis_name="c", subcore_axis_name="s")

    @functools.partial(
        pl.kernel,
        mesh=mesh,
        out_type=jax.ShapeDtypeStruct((10, BATCH), jnp.float32),
        compiler_params=pltpu.CompilerParams(
            use_tc_tiling_on_sc=False, needs_layout_passes=False),
        scratch_types=[
            pltpu.VMEM((BPW,), jnp.int32),
            pltpu.VMEM((BPW,), jnp.int32),
            pltpu.VMEM((10 * BPW, 8), jnp.float32),
            pltpu.VMEM((10, BPW), jnp.float32),
            pltpu.SemaphoreType.DMA,
        ],
    )
    def k(intr_hbm, rot_hbm, trans_hbm, idx_hbm, out_hbm,
          idx_v, gl, vbuf, out_v, sem):
        wid = lax.axis_index("s") * NC + lax.axis_index("c")
        base = wid * BPW
        pltpu.sync_copy(idx_hbm.at[pl.ds(base, BPW)], idx_v)

        for ch in range(BPW // 16):
            iv = idx_v[pl.ds(ch * 16, 16)]
            gl[pl.ds(ch * 16, 16)] = lax.shift_right_logical(iv, 3)

        sources = ([intr_hbm.at[pl.ds(c * CW, CW), :] for c in range(4)]
                   + [rot_hbm.at[pl.ds(c * CW, CW), :] for c in range(3)]
                   + [trans_hbm.at[pl.ds(c * CW, CW), :] for c in range(3)])
        copies = []
        for q, src in enumerate(sources):
            copies.append(pltpu.async_copy(
                src.at[gl], vbuf.at[pl.ds(q * BPW, BPW), :], sem))
        for cpy in copies:
            cpy.wait()

        iota = lax.iota(jnp.int32, 16)

        def body(chunk, carry):
            rows = chunk * 16 + iota
            iv = idx_v[pl.ds(chunk * 16, 16)]
            off = iv & 7
            for q in range(10):
                vals = plsc.load_gather(vbuf, [q * BPW + rows, off])
                plsc.store_scatter(out_v, [jnp.full((16,), q, jnp.int32), rows],
                                   vals)
            return carry

        lax.fori_loop(0, BPW // 16, body, 0)
        pltpu.sync_copy(out_v, out_hbm.at[:, pl.ds(base, BPW)])

    return k(intr8, rot8, trans8, idx).T


def kernel(intrinsic_deltas, rotation_deltas, translation_deltas, camera_idx):
    intr8 = intrinsic_deltas.T.reshape(4 * NROW // 8, 8)
    rot8 = rotation_deltas.T.reshape(3 * NROW // 8, 8)
    trans8 = translation_deltas.T.reshape(3 * NROW // 8, 8)
    return _sc_gather_concat(intr8, rot8, trans8,
                             camera_idx.astype(jnp.int32))
